# Initial kernel scaffold; baseline (speedup 1.0000x reference)
import jax
import jax.numpy as jnp
from jax.experimental import pallas as pl


def kernel(token_ids, weight):
    raise NotImplementedError("write your pallas kernel here")


# speedup vs baseline: 1.0236x; 1.0236x over previous
import functools

import jax
import jax.numpy as jnp
from jax import lax
from jax.experimental import pallas as pl
from jax.experimental.pallas import tpu as pltpu
from jax.experimental.pallas import tpu_sc as plsc

NUM_EMB = 1000000
DIM = 32
ROWS = 16384 * 50
NC, NS = 2, 16
NW = NC * NS
PER_W = ROWS // NW
K = 128
NCHUNK = PER_W // K

_MESH = plsc.VectorSubcoreMesh(
    core_axis_name="c", subcore_axis_name="s", num_cores=NC, num_subcores=NS
)


@functools.partial(
    pl.kernel,
    out_type=jax.ShapeDtypeStruct((ROWS, DIM), jnp.float32),
    mesh=_MESH,
    scratch_types=[
        pltpu.VMEM((NCHUNK, K), jnp.int32),
        pltpu.VMEM((K, DIM), jnp.float32),
        pltpu.SemaphoreType.DMA,
    ],
    compiler_params=pltpu.CompilerParams(use_tc_tiling_on_sc=False),
)
def _emb_lookup(idx_hbm, table_hbm, out_hbm, idx_v, rows_v, sem):
    wid = lax.axis_index("s") * NC + lax.axis_index("c")
    base = wid * PER_W
    pltpu.sync_copy(idx_hbm.at[wid], idx_v)

    def body(c, carry):
        pltpu.async_copy(table_hbm.at[idx_v.at[c]], rows_v, sem).wait()
        pltpu.sync_copy(rows_v, out_hbm.at[pl.ds(base + c * K, K)])
        return carry

    lax.fori_loop(0, NCHUNK, body, 0)


def kernel(token_ids, weight):
    idx = token_ids.reshape(NW, NCHUNK, K).astype(jnp.int32)
    out = _emb_lookup(idx, weight)
    return out.reshape(token_ids.shape[0], token_ids.shape[1], DIM)


# speedup vs baseline: 1.1125x; 1.0869x over previous
import functools

import jax
import jax.numpy as jnp
from jax import lax
from jax.experimental import pallas as pl
from jax.experimental.pallas import tpu as pltpu
from jax.experimental.pallas import tpu_sc as plsc

NUM_EMB = 1000000
DIM = 32
ROWS = 16384 * 50
NC, NS = 2, 16
NW = NC * NS
PER_W = ROWS // NW
K = 128
NCHUNK = PER_W // K
NBUF = 8
NITER = NCHUNK // NBUF

_MESH = plsc.VectorSubcoreMesh(
    core_axis_name="c", subcore_axis_name="s", num_cores=NC, num_subcores=NS
)


@functools.partial(
    pl.kernel,
    out_type=jax.ShapeDtypeStruct((ROWS, DIM), jnp.float32),
    mesh=_MESH,
    scratch_types=[
        pltpu.VMEM((NCHUNK, K), jnp.int32),
        pltpu.VMEM((NBUF, K, DIM), jnp.float32),
        pltpu.SemaphoreType.DMA((NBUF,)),
        pltpu.SemaphoreType.DMA((NBUF,)),
    ],
    compiler_params=pltpu.CompilerParams(use_tc_tiling_on_sc=False),
)
def _emb_lookup(idx_hbm, table_hbm, out_hbm, idx_v, rows_v, gsem, ssem):
    wid = lax.axis_index("s") * NC + lax.axis_index("c")
    base = wid * PER_W
    pltpu.sync_copy(idx_hbm.at[wid], idx_v)

    def gather(c, b):
        return pltpu.make_async_copy(
            table_hbm.at[idx_v.at[c]], rows_v.at[b], gsem.at[b]
        )

    def store(c, b):
        return pltpu.make_async_copy(
            rows_v.at[b], out_hbm.at[pl.ds(base + c * K, K)], ssem.at[b]
        )

    for b in range(NBUF):
        gather(b, b).start()

    def body(it, carry):
        g = it * NBUF
        for b in range(NBUF):
            gather(g + b, b).wait()
            store(g + b, b).start()
        for b in range(NBUF):
            c_next = g + NBUF + b

            @pl.when(c_next < NCHUNK)
            def _():
                store(g + b, b).wait()
                gather(c_next, b).start()

        return carry

    lax.fori_loop(0, NITER, body, 0)

    for b in range(NBUF):
        store(NCHUNK - NBUF + b, b).wait()


def kernel(token_ids, weight):
    idx = token_ids.reshape(NW, NCHUNK, K).astype(jnp.int32)
    out = _emb_lookup(idx, weight)
    return out.reshape(token_ids.shape[0], token_ids.shape[1], DIM)


# speedup vs baseline: 1.8143x; 1.6308x over previous
import functools

import jax
import jax.numpy as jnp
from jax import lax
from jax.experimental import pallas as pl
from jax.experimental.pallas import tpu as pltpu
from jax.experimental.pallas import tpu_sc as plsc

NUM_EMB = 1000000
DIM = 32
BATCH = 16384
SEQ = 50
NC, NS = 2, 16
NW = NC * NS
BPW = BATCH // NW
K = 128
JB = BPW // K
NCHUNK = SEQ * JB
NBUF = 8
NITER = NCHUNK // NBUF

_MESH = plsc.VectorSubcoreMesh(
    core_axis_name="c", subcore_axis_name="s", num_cores=NC, num_subcores=NS
)


@functools.partial(
    pl.kernel,
    out_type=jax.ShapeDtypeStruct((BATCH, SEQ, DIM), jnp.float32),
    mesh=_MESH,
    scratch_types=[
        pltpu.VMEM((SEQ, BPW), jnp.int32),
        pltpu.VMEM((NBUF, K, DIM), jnp.float32),
        pltpu.SemaphoreType.DMA((NBUF,)),
        pltpu.SemaphoreType.DMA((NBUF,)),
    ],
    compiler_params=pltpu.CompilerParams(use_tc_tiling_on_sc=False),
)
def _emb_lookup(tok_hbm, table_hbm, out_hbm, idx_v, rows_v, gsem, ssem):
    wid = lax.axis_index("s") * NC + lax.axis_index("c")
    b0 = wid * BPW
    pltpu.sync_copy(tok_hbm.at[:, pl.ds(b0, BPW)], idx_v)

    def gather(c, b):
        t, j = c // JB, c % JB
        return pltpu.make_async_copy(
            table_hbm.at[idx_v.at[t, pl.ds(j * K, K)]], rows_v.at[b], gsem.at[b]
        )

    def store(c, b):
        t, j = c // JB, c % JB
        return pltpu.make_async_copy(
            rows_v.at[b], out_hbm.at[pl.ds(b0 + j * K, K), t], ssem.at[b]
        )

    for b in range(NBUF):
        gather(b, b).start()

    def body(it, carry):
        g = it * NBUF
        for b in range(NBUF):
            gather(g + b, b).wait()
            store(g + b, b).start()
        for b in range(NBUF):
            c_next = g + NBUF + b

            @pl.when(c_next < NCHUNK)
            def _():
                store(g + b, b).wait()
                gather(c_next, b).start()

        return carry

    lax.fori_loop(0, NITER, body, 0)

    for b in range(NBUF):
        store(NCHUNK - NBUF + b, b).wait()


def kernel(token_ids, weight):
    return _emb_lookup(token_ids.T, weight)
